# baseline (speedup 1.0000x reference)
import functools

import jax
import jax.numpy as jnp
from jax import lax
from jax.experimental import pallas as pl
from jax.experimental.pallas import tpu as pltpu
from jax.experimental.pallas import tpu_sc as plsc

RADIUS = 0.1
INV_R = 1.0 / RADIUS
L = 16
NW = 32
QB = 4
CH = 112
M = 32
TW = 64
D = 48
RW = 1568


def _bcast(v, k):
  idx = jnp.full((L, 1), k, dtype=jnp.int32)
  return lax.gather(
      v, idx,
      dimension_numbers=lax.GatherDimensionNumbers(
          offset_dims=(), collapsed_slice_dims=(0,), start_index_map=(0,)),
      slice_sizes=(1,),
      mode=lax.GatherScatterMode.PROMISE_IN_BOUNDS)


def _sc_body(n, table, nbf, qpf, pm, out, st_out, idx_a, qp_a, pm_v,
             rows_a, rows_b, rows_c, rows_d, out_c, st_v,
             sem_a, sem_b, sem_c, sem_d):
  rl = n - (NW - 1) * RW
  c_id = lax.axis_index("c")
  s_id = lax.axis_index("s")
  wid = s_id * 2 + c_id
  is_last = wid == NW - 1
  row0 = wid * RW
  rw = jnp.where(is_last, rl, RW)
  nblk = rw // QB
  bc = CH // QB

  @pl.when(jnp.logical_not(is_last))
  def _():
    pltpu.sync_copy(nbf.at[pl.ds(row0 * M, RW * M)], idx_a.at[pl.ds(0, RW * M)])
    pltpu.sync_copy(qpf.at[pl.ds(row0 * 16, RW * 16)],
                    qp_a.at[pl.ds(0, RW * 16)])

  @pl.when(is_last)
  def _():
    pltpu.sync_copy(nbf.at[pl.ds(row0 * M, (n - (NW - 1) * RW) * M)],
                    idx_a.at[pl.ds(0, rl * M)])
    pltpu.sync_copy(qpf.at[pl.ds(row0 * 16, rl * 16)],
                    qp_a.at[pl.ds(0, rl * 16)])

  pltpu.sync_copy(pm, pm_v)
  pmv = pm_v[...]

  for j in range(6):
    st_v[pl.ds(j * L, L)] = jnp.zeros((L,), jnp.float32)

  def gather(b, buf, sem):
    pltpu.async_copy(table.at[idx_a.at[pl.ds(b * QB * M, QB * M)]], buf, sem)

  def drain(buf, sem):
    pltpu.make_async_copy(table.at[idx_a.at[pl.ds(0, QB * M)]], buf,
                          sem).wait()

  def compute(b, buf):
    ob = lax.rem(b, bc) * QB
    for r in range(QB):
      rr = b * QB + r
      orow = ob + r
      qpv = qp_a[pl.ds(rr * 16, L)]
      qx = _bcast(qpv, 0)
      qy = _bcast(qpv, 1)
      qz = _bcast(qpv, 2)
      acc = [jnp.zeros((L,), jnp.float32) for _ in range(6)]
      for m in range(M):
        row = r * M + m
        f0, f1 = plsc.unpack(buf[row, pl.ds(0, 2 * L)],
                             format=plsc.PackFormat.INTERLEAVED)
        f2, aux = plsc.unpack(buf[row, pl.ds(2 * L, 2 * L)],
                              format=plsc.PackFormat.INTERLEAVED)
        p = 3 * (m % 2)
        acc[p + 0] = acc[p + 0] + (_bcast(aux, 0) - qx) * f0
        acc[p + 1] = acc[p + 1] + (_bcast(aux, 1) - qy) * f1
        acc[p + 2] = acc[p + 2] + (_bcast(aux, 2) - qz) * f2
      i0 = idx_a[pl.ds(rr * M, L)]
      i1 = idx_a[pl.ds(rr * M + L, L)]
      c0 = plsc.all_reduce_population_count(i0 < pmv)
      c1 = plsc.all_reduce_population_count(i1 < pmv)
      cnt = (c0 + c1).astype(jnp.float32) + 1e-5
      scale = INV_R / cnt
      a = [(acc[j] + acc[3 + j]) * scale for j in range(3)]
      for j in range(3):
        out_c[pl.ds(orow * D + j * L, L)] = a[j]
        st_v[pl.ds(j * L, L)] += a[j]
        st_v[pl.ds((3 + j) * L, L)] += a[j] * a[j]

  bufs = [rows_a, rows_b]
  sems = [sem_a, sem_b]
  nd = 2

  for j in range(nd - 1):
    gather(j, bufs[j], sems[j])

  def body(i, _):
    b0 = nd * i
    for j in range(nd):
      b = b0 + j
      nxt = b + nd - 1

      @pl.when(nxt < nblk)
      def _():
        gather(nxt, bufs[(j + nd - 1) % nd], sems[(j + nd - 1) % nd])

      drain(bufs[j], sems[j])
      compute(b, bufs[j])

      @pl.when(lax.rem(b, bc) == bc - 1)
      def _():
        pltpu.sync_copy(out_c,
                        out.at[pl.ds((row0 + lax.div(b, bc) * CH) * D,
                                     CH * D)])

    return ()

  lax.fori_loop(0, nblk // nd, body, ())

  tail = rl % CH
  if tail:
    @pl.when(is_last)
    def _():
      pltpu.sync_copy(
          out_c.at[pl.ds(0, tail * D)],
          out.at[pl.ds((row0 + (rl // CH) * CH) * D, tail * D)])

  pltpu.sync_copy(st_v, st_out.at[pl.ds(wid * 6 * L, 6 * L)])


@functools.partial(jax.jit, static_argnums=(4,))
def _sc_call(table, nbf, qpf, pm, n):
  mesh = plsc.VectorSubcoreMesh(core_axis_name="c", subcore_axis_name="s")
  return pl.kernel(
      functools.partial(_sc_body, n),
      out_type=(jax.ShapeDtypeStruct((n * D,), jnp.float32),
                jax.ShapeDtypeStruct((NW * 6 * L,), jnp.float32)),
      mesh=mesh,
      scratch_types=[
          pltpu.VMEM((RW * M,), jnp.int32),
          pltpu.VMEM((RW * 16,), jnp.float32),
          pltpu.VMEM((L,), jnp.int32),
          pltpu.VMEM((QB * M, TW), jnp.bfloat16),
          pltpu.VMEM((QB * M, TW), jnp.bfloat16),
          pltpu.VMEM((QB * M, TW), jnp.bfloat16),
          pltpu.VMEM((QB * M, TW), jnp.bfloat16),
          pltpu.VMEM((CH * D,), jnp.float32),
          pltpu.VMEM((6 * L,), jnp.float32),
          pltpu.SemaphoreType.DMA,
          pltpu.SemaphoreType.DMA,
          pltpu.SemaphoreType.DMA,
          pltpu.SemaphoreType.DMA,
      ],
      compiler_params=pltpu.CompilerParams(use_tc_tiling_on_sc=False,
                                           needs_layout_passes=False),
  )(table, nbf, qpf, pm)


def _max_body(nb_ref, out_ref):
  i = pl.program_id(0)

  @pl.when(i == 0)
  def _():
    out_ref[0, 0] = jnp.int32(-2**31)

  out_ref[0, 0] = jnp.maximum(out_ref[0, 0], jnp.max(nb_ref[...]))


def _norm_body(n_rows, st_ref, w_ref, b_ref, a_ref, o_ref):
  s = jnp.sum(st_ref[...], axis=0, keepdims=True)
  mean = s[:, :D] / n_rows
  var = s[:, D:] / n_rows - mean * mean
  inv = lax.rsqrt(var + 1e-5)
  y = (a_ref[...] - mean) * inv * w_ref[...] + b_ref[...]
  o_ref[...] = jnp.maximum(y, 0.2 * y)


def kernel(query_points, support_points, neighbors, x, bn_weight, bn_bias):
  n, d = x.shape
  assert d == D and n == (NW - 1) * RW + (n - (NW - 1) * RW)
  nb = neighbors.astype(jnp.int32)

  xb = x.astype(jnp.bfloat16)
  aux = jnp.concatenate(
      [support_points.astype(jnp.bfloat16),
       jnp.zeros((n, L - 3), jnp.bfloat16)], axis=1)
  first = jnp.stack([xb[:, :L], xb[:, L:2 * L]], axis=2).reshape(n, 2 * L)
  second = jnp.stack([xb[:, 2 * L:D], aux], axis=2).reshape(n, 2 * L)
  table = jnp.concatenate([first, second], axis=1)
  nbf = nb.reshape(-1)
  qpf = jnp.pad(query_points, ((0, 0), (0, 16 - 3))).reshape(-1)

  blk = 2000
  grid = n // blk
  padmax = pl.pallas_call(
      _max_body,
      grid=(grid,),
      in_specs=[pl.BlockSpec((blk, M), lambda i: (i, 0))],
      out_specs=pl.BlockSpec((1, 1), lambda i: (0, 0),
                             memory_space=pltpu.SMEM),
      out_shape=jax.ShapeDtypeStruct((1, 1), jnp.int32),
  )(nb)
  pm16 = jnp.broadcast_to(padmax.reshape(1), (L,))

  out_flat, st = _sc_call(table, nbf, qpf, pm16, n)
  a2 = out_flat.reshape(n, D)
  st2 = st.reshape(NW, 6 * L)

  out = pl.pallas_call(
      functools.partial(_norm_body, float(n)),
      grid=(grid,),
      in_specs=[
          pl.BlockSpec((NW, 6 * L), lambda i: (0, 0)),
          pl.BlockSpec((1, D), lambda i: (0, 0)),
          pl.BlockSpec((1, D), lambda i: (0, 0)),
          pl.BlockSpec((blk, D), lambda i: (i, 0)),
      ],
      out_specs=pl.BlockSpec((blk, D), lambda i: (i, 0)),
      out_shape=jax.ShapeDtypeStruct((n, D), jnp.float32),
  )(st2, bn_weight.reshape(1, D), bn_bias.reshape(1, D), a2)

  return out


# speedup vs baseline: 1.5291x; 1.5291x over previous
import functools

import jax
import jax.numpy as jnp
from jax import lax
from jax.experimental import pallas as pl
from jax.experimental.pallas import tpu as pltpu
from jax.experimental.pallas import tpu_sc as plsc

RADIUS = 0.1
INV_R = 1.0 / RADIUS
L = 16
NW = 32
QB = 4
CH = 112
M = 32
TW = 64
D = 48
RW = 1568


def _bcast(v, k):
  idx = jnp.full((L, 1), k, dtype=jnp.int32)
  return lax.gather(
      v, idx,
      dimension_numbers=lax.GatherDimensionNumbers(
          offset_dims=(), collapsed_slice_dims=(0,), start_index_map=(0,)),
      slice_sizes=(1,),
      mode=lax.GatherScatterMode.PROMISE_IN_BOUNDS)


def _sc_body(n, table, nbf, qpf, pm, out, st_out, idx_a, qp_a, pm_v,
             rows_a, rows_b, rows_c, rows_d, out_c, st_v,
             sem_a, sem_b, sem_c, sem_d):
  rl = n - (NW - 1) * RW
  c_id = lax.axis_index("c")
  s_id = lax.axis_index("s")
  wid = s_id * 2 + c_id
  is_last = wid == NW - 1
  row0 = wid * RW
  rw = jnp.where(is_last, rl, RW)
  nblk = rw // QB
  bc = CH // QB

  @pl.when(jnp.logical_not(is_last))
  def _():
    pltpu.sync_copy(nbf.at[pl.ds(row0 * M, RW * M)], idx_a.at[pl.ds(0, RW * M)])
    pltpu.sync_copy(qpf.at[pl.ds(row0 * 16, RW * 16)],
                    qp_a.at[pl.ds(0, RW * 16)])

  @pl.when(is_last)
  def _():
    pltpu.sync_copy(nbf.at[pl.ds(row0 * M, (n - (NW - 1) * RW) * M)],
                    idx_a.at[pl.ds(0, rl * M)])
    pltpu.sync_copy(qpf.at[pl.ds(row0 * 16, rl * 16)],
                    qp_a.at[pl.ds(0, rl * 16)])

  pltpu.sync_copy(pm, pm_v)
  pmv = pm_v[...]

  for j in range(6):
    st_v[pl.ds(j * L, L)] = jnp.zeros((L,), jnp.float32)

  def gather(b, buf, sem):
    pltpu.async_copy(table.at[idx_a.at[pl.ds(b * QB * M, QB * M)]], buf, sem)

  def drain(buf, sem):
    pltpu.make_async_copy(table.at[idx_a.at[pl.ds(0, QB * M)]], buf,
                          sem).wait()

  def compute(b, buf):
    ob = lax.rem(b, bc) * QB
    for r in range(QB):
      rr = b * QB + r
      orow = ob + r
      qpv = qp_a[pl.ds(rr * 16, L)]
      qx = _bcast(qpv, 0)
      qy = _bcast(qpv, 1)
      qz = _bcast(qpv, 2)
      acc = [jnp.zeros((L,), jnp.float32) for _ in range(6)]
      for m in range(M):
        row = r * M + m
        xyz = buf[row, pl.ds(D, L)]
        f0 = buf[row, pl.ds(0, L)]
        f1 = buf[row, pl.ds(L, L)]
        f2 = buf[row, pl.ds(2 * L, L)]
        p = 3 * (m % 2)
        acc[p + 0] = acc[p + 0] + (_bcast(xyz, 0) - qx) * f0
        acc[p + 1] = acc[p + 1] + (_bcast(xyz, 1) - qy) * f1
        acc[p + 2] = acc[p + 2] + (_bcast(xyz, 2) - qz) * f2
      i0 = idx_a[pl.ds(rr * M, L)]
      i1 = idx_a[pl.ds(rr * M + L, L)]
      c0 = plsc.all_reduce_population_count(i0 < pmv)
      c1 = plsc.all_reduce_population_count(i1 < pmv)
      cnt = (c0 + c1).astype(jnp.float32) + 1e-5
      scale = INV_R / cnt
      a = [(acc[j] + acc[3 + j]) * scale for j in range(3)]
      for j in range(3):
        out_c[pl.ds(orow * D + j * L, L)] = a[j]
        st_v[pl.ds(j * L, L)] += a[j]
        st_v[pl.ds((3 + j) * L, L)] += a[j] * a[j]

  bufs = [rows_a, rows_b]
  sems = [sem_a, sem_b]
  nd = 2

  for j in range(nd - 1):
    gather(j, bufs[j], sems[j])

  def body(i, _):
    b0 = nd * i
    for j in range(nd):
      b = b0 + j
      nxt = b + nd - 1

      @pl.when(nxt < nblk)
      def _():
        gather(nxt, bufs[(j + nd - 1) % nd], sems[(j + nd - 1) % nd])

      drain(bufs[j], sems[j])
      compute(b, bufs[j])

      @pl.when(lax.rem(b, bc) == bc - 1)
      def _():
        pltpu.sync_copy(out_c,
                        out.at[pl.ds((row0 + lax.div(b, bc) * CH) * D,
                                     CH * D)])

    return ()

  lax.fori_loop(0, nblk // nd, body, ())

  tail = rl % CH
  if tail:
    @pl.when(is_last)
    def _():
      pltpu.sync_copy(
          out_c.at[pl.ds(0, tail * D)],
          out.at[pl.ds((row0 + (rl // CH) * CH) * D, tail * D)])

  pltpu.sync_copy(st_v, st_out.at[pl.ds(wid * 6 * L, 6 * L)])


@functools.partial(jax.jit, static_argnums=(4,))
def _sc_call(table, nbf, qpf, pm, n):
  mesh = plsc.VectorSubcoreMesh(core_axis_name="c", subcore_axis_name="s")
  return pl.kernel(
      functools.partial(_sc_body, n),
      out_type=(jax.ShapeDtypeStruct((n * D,), jnp.float32),
                jax.ShapeDtypeStruct((NW * 6 * L,), jnp.float32)),
      mesh=mesh,
      scratch_types=[
          pltpu.VMEM((RW * M,), jnp.int32),
          pltpu.VMEM((RW * 16,), jnp.float32),
          pltpu.VMEM((L,), jnp.int32),
          pltpu.VMEM((QB * M, TW), jnp.float32),
          pltpu.VMEM((QB * M, TW), jnp.float32),
          pltpu.VMEM((QB * M, TW), jnp.float32),
          pltpu.VMEM((QB * M, TW), jnp.float32),
          pltpu.VMEM((CH * D,), jnp.float32),
          pltpu.VMEM((6 * L,), jnp.float32),
          pltpu.SemaphoreType.DMA,
          pltpu.SemaphoreType.DMA,
          pltpu.SemaphoreType.DMA,
          pltpu.SemaphoreType.DMA,
      ],
      compiler_params=pltpu.CompilerParams(use_tc_tiling_on_sc=False,
                                           needs_layout_passes=False),
  )(table, nbf, qpf, pm)


def _max_body(nb_ref, out_ref):
  i = pl.program_id(0)

  @pl.when(i == 0)
  def _():
    out_ref[0, 0] = jnp.int32(-2**31)

  out_ref[0, 0] = jnp.maximum(out_ref[0, 0], jnp.max(nb_ref[...]))


def _norm_body(n_rows, st_ref, w_ref, b_ref, a_ref, o_ref):
  s = jnp.sum(st_ref[...], axis=0, keepdims=True)
  mean = s[:, :D] / n_rows
  var = s[:, D:] / n_rows - mean * mean
  inv = lax.rsqrt(var + 1e-5)
  y = (a_ref[...] - mean) * inv * w_ref[...] + b_ref[...]
  o_ref[...] = jnp.maximum(y, 0.2 * y)


def kernel(query_points, support_points, neighbors, x, bn_weight, bn_bias):
  n, d = x.shape
  assert d == D and n == (NW - 1) * RW + (n - (NW - 1) * RW)
  nb = neighbors.astype(jnp.int32)

  table_flat = jnp.concatenate(
      [x, support_points,
       jnp.zeros((n, TW - D - 3), jnp.float32)], axis=1).reshape(-1)
  table_flat = lax.optimization_barrier(table_flat)
  table = table_flat.reshape(n, TW)
  nbf = nb.reshape(-1)
  qpf = jnp.pad(query_points, ((0, 0), (0, 16 - 3))).reshape(-1)

  blk = 2000
  grid = n // blk
  padmax = pl.pallas_call(
      _max_body,
      grid=(grid,),
      in_specs=[pl.BlockSpec((blk, M), lambda i: (i, 0))],
      out_specs=pl.BlockSpec((1, 1), lambda i: (0, 0),
                             memory_space=pltpu.SMEM),
      out_shape=jax.ShapeDtypeStruct((1, 1), jnp.int32),
  )(nb)
  pm16 = jnp.broadcast_to(padmax.reshape(1), (L,))

  out_flat, st = _sc_call(table, nbf, qpf, pm16, n)
  a2 = out_flat.reshape(n, D)
  st2 = st.reshape(NW, 6 * L)

  out = pl.pallas_call(
      functools.partial(_norm_body, float(n)),
      grid=(grid,),
      in_specs=[
          pl.BlockSpec((NW, 6 * L), lambda i: (0, 0)),
          pl.BlockSpec((1, D), lambda i: (0, 0)),
          pl.BlockSpec((1, D), lambda i: (0, 0)),
          pl.BlockSpec((blk, D), lambda i: (i, 0)),
      ],
      out_specs=pl.BlockSpec((blk, D), lambda i: (i, 0)),
      out_shape=jax.ShapeDtypeStruct((n, D), jnp.float32),
  )(st2, bn_weight.reshape(1, D), bn_bias.reshape(1, D), a2)

  return out


# speedup vs baseline: 1.5292x; 1.0001x over previous
import functools

import jax
import jax.numpy as jnp
from jax import lax
from jax.experimental import pallas as pl
from jax.experimental.pallas import tpu as pltpu
from jax.experimental.pallas import tpu_sc as plsc

RADIUS = 0.1
INV_R = 1.0 / RADIUS
L = 16
NW = 32
QB = 4
CH = 112
M = 32
TW = 64
D = 48
RW = 1568


def _bcast(v, k):
  idx = jnp.full((L, 1), k, dtype=jnp.int32)
  return lax.gather(
      v, idx,
      dimension_numbers=lax.GatherDimensionNumbers(
          offset_dims=(), collapsed_slice_dims=(0,), start_index_map=(0,)),
      slice_sizes=(1,),
      mode=lax.GatherScatterMode.PROMISE_IN_BOUNDS)


def _sc_body(n, table, nbf, qpf, pm, out, st_out, idx_a, qp_a, pm_v,
             rows_a, rows_b, rows_c, rows_d, out_c, st_v,
             sem_a, sem_b, sem_c, sem_d):
  rl = n - (NW - 1) * RW
  c_id = lax.axis_index("c")
  s_id = lax.axis_index("s")
  wid = s_id * 2 + c_id
  is_last = wid == NW - 1
  row0 = wid * RW
  rw = jnp.where(is_last, rl, RW)
  nblk = rw // QB
  bc = CH // QB

  @pl.when(jnp.logical_not(is_last))
  def _():
    pltpu.sync_copy(nbf.at[pl.ds(row0 * M, RW * M)], idx_a.at[pl.ds(0, RW * M)])
    pltpu.sync_copy(qpf.at[pl.ds(row0 * 16, RW * 16)],
                    qp_a.at[pl.ds(0, RW * 16)])

  @pl.when(is_last)
  def _():
    pltpu.sync_copy(nbf.at[pl.ds(row0 * M, (n - (NW - 1) * RW) * M)],
                    idx_a.at[pl.ds(0, rl * M)])
    pltpu.sync_copy(qpf.at[pl.ds(row0 * 16, rl * 16)],
                    qp_a.at[pl.ds(0, rl * 16)])

  pltpu.sync_copy(pm, pm_v)
  pmv = pm_v[...]

  for j in range(6):
    st_v[pl.ds(j * L, L)] = jnp.zeros((L,), jnp.float32)

  def gather(b, buf, sem):
    pltpu.async_copy(table.at[idx_a.at[pl.ds(b * QB * M, QB * M)]], buf, sem)

  def drain(buf, sem):
    pltpu.make_async_copy(table.at[idx_a.at[pl.ds(0, QB * M)]], buf,
                          sem).wait()

  def compute(b, buf):
    ob = lax.rem(b, bc) * QB
    for r in range(QB):
      rr = b * QB + r
      orow = ob + r
      qpv = qp_a[pl.ds(rr * 16, L)]
      qx = _bcast(qpv, 0)
      qy = _bcast(qpv, 1)
      qz = _bcast(qpv, 2)
      acc = [jnp.zeros((L,), jnp.float32) for _ in range(6)]
      for m in range(M):
        row = r * M + m
        xyz = buf[row, pl.ds(D, L)]
        f0 = buf[row, pl.ds(0, L)]
        f1 = buf[row, pl.ds(L, L)]
        f2 = buf[row, pl.ds(2 * L, L)]
        p = 3 * (m % 2)
        acc[p + 0] = acc[p + 0] + (_bcast(xyz, 0) - qx) * f0
        acc[p + 1] = acc[p + 1] + (_bcast(xyz, 1) - qy) * f1
        acc[p + 2] = acc[p + 2] + (_bcast(xyz, 2) - qz) * f2
      i0 = idx_a[pl.ds(rr * M, L)]
      i1 = idx_a[pl.ds(rr * M + L, L)]
      c0 = plsc.all_reduce_population_count(i0 < pmv)
      c1 = plsc.all_reduce_population_count(i1 < pmv)
      cnt = (c0 + c1).astype(jnp.float32) + 1e-5
      scale = INV_R / cnt
      a = [(acc[j] + acc[3 + j]) * scale for j in range(3)]
      for j in range(3):
        out_c[pl.ds(orow * D + j * L, L)] = a[j]
        st_v[pl.ds(j * L, L)] += a[j]
        st_v[pl.ds((3 + j) * L, L)] += a[j] * a[j]

  bufs = [rows_a, rows_b]
  sems = [sem_a, sem_b]
  nd = 2

  for j in range(nd - 1):
    gather(j, bufs[j], sems[j])

  def body(i, _):
    b0 = nd * i
    for j in range(nd):
      b = b0 + j
      nxt = b + nd - 1

      @pl.when(nxt < nblk)
      def _():
        gather(nxt, bufs[(j + nd - 1) % nd], sems[(j + nd - 1) % nd])

      drain(bufs[j], sems[j])
      compute(b, bufs[j])

      @pl.when(lax.rem(b, bc) == bc - 1)
      def _():
        pltpu.sync_copy(out_c,
                        out.at[pl.ds((row0 + lax.div(b, bc) * CH) * D,
                                     CH * D)])

    return ()

  lax.fori_loop(0, nblk // nd, body, ())

  tail = rl % CH
  if tail:
    @pl.when(is_last)
    def _():
      pltpu.sync_copy(
          out_c.at[pl.ds(0, tail * D)],
          out.at[pl.ds((row0 + (rl // CH) * CH) * D, tail * D)])

  pltpu.sync_copy(st_v, st_out.at[pl.ds(wid * 6 * L, 6 * L)])


@functools.partial(jax.jit, static_argnums=(4,))
def _sc_call(table, nbf, qpf, pm, n):
  mesh = plsc.VectorSubcoreMesh(core_axis_name="c", subcore_axis_name="s")
  return pl.kernel(
      functools.partial(_sc_body, n),
      out_type=(jax.ShapeDtypeStruct((n * D,), jnp.float32),
                jax.ShapeDtypeStruct((NW * 6 * L,), jnp.float32)),
      mesh=mesh,
      scratch_types=[
          pltpu.VMEM((RW * M,), jnp.int32),
          pltpu.VMEM((RW * 16,), jnp.float32),
          pltpu.VMEM((L,), jnp.int32),
          pltpu.VMEM((QB * M, TW), jnp.float32),
          pltpu.VMEM((QB * M, TW), jnp.float32),
          pltpu.VMEM((QB * M, TW), jnp.float32),
          pltpu.VMEM((QB * M, TW), jnp.float32),
          pltpu.VMEM((CH * D,), jnp.float32),
          pltpu.VMEM((6 * L,), jnp.float32),
          pltpu.SemaphoreType.DMA,
          pltpu.SemaphoreType.DMA,
          pltpu.SemaphoreType.DMA,
          pltpu.SemaphoreType.DMA,
      ],
      compiler_params=pltpu.CompilerParams(use_tc_tiling_on_sc=False,
                                           needs_layout_passes=False),
  )(table, nbf, qpf, pm)


def _max_body(nb_ref, out_ref):
  i = pl.program_id(0)

  @pl.when(i == 0)
  def _():
    out_ref[0, 0] = jnp.int32(-2**31)

  out_ref[0, 0] = jnp.maximum(out_ref[0, 0], jnp.max(nb_ref[...]))


def _norm_body(n_rows, st_ref, w_ref, b_ref, a_ref, o_ref):
  s = jnp.sum(st_ref[...], axis=0, keepdims=True)
  mean = s[:, :D] / n_rows
  var = s[:, D:] / n_rows - mean * mean
  inv = lax.rsqrt(var + 1e-5)
  y = (a_ref[...] - mean) * inv * w_ref[...] + b_ref[...]
  o_ref[...] = jnp.maximum(y, 0.2 * y)


def kernel(query_points, support_points, neighbors, x, bn_weight, bn_bias):
  n, d = x.shape
  assert d == D and n == (NW - 1) * RW + (n - (NW - 1) * RW)
  nb = neighbors.astype(jnp.int32)

  table_flat = jnp.concatenate(
      [x, support_points,
       jnp.zeros((n, TW - D - 3), jnp.float32)], axis=1).reshape(-1)
  table_flat = lax.optimization_barrier(table_flat)
  table = table_flat.reshape(n, TW)
  nbf = lax.optimization_barrier(nb.reshape(-1))
  qpf = lax.optimization_barrier(
      jnp.pad(query_points, ((0, 0), (0, 16 - 3))).reshape(-1))

  blk = 2000
  grid = n // blk
  padmax = pl.pallas_call(
      _max_body,
      grid=(grid,),
      in_specs=[pl.BlockSpec((blk, M), lambda i: (i, 0))],
      out_specs=pl.BlockSpec((1, 1), lambda i: (0, 0),
                             memory_space=pltpu.SMEM),
      out_shape=jax.ShapeDtypeStruct((1, 1), jnp.int32),
  )(nb)
  pm16 = jnp.broadcast_to(padmax.reshape(1), (L,))

  out_flat, st = _sc_call(table, nbf, qpf, pm16, n)
  a2 = out_flat.reshape(n, D)
  st2 = st.reshape(NW, 6 * L)

  out = pl.pallas_call(
      functools.partial(_norm_body, float(n)),
      grid=(grid,),
      in_specs=[
          pl.BlockSpec((NW, 6 * L), lambda i: (0, 0)),
          pl.BlockSpec((1, D), lambda i: (0, 0)),
          pl.BlockSpec((1, D), lambda i: (0, 0)),
          pl.BlockSpec((blk, D), lambda i: (i, 0)),
      ],
      out_specs=pl.BlockSpec((blk, D), lambda i: (i, 0)),
      out_shape=jax.ShapeDtypeStruct((n, D), jnp.float32),
  )(st2, bn_weight.reshape(1, D), bn_bias.reshape(1, D), a2)

  return out
